# baseline (speedup 1.0000x reference)
import functools

import jax
import jax.numpy as jnp
from jax import lax
from jax.experimental import pallas as pl
from jax.experimental.pallas import tpu as pltpu
from jax.experimental.pallas import tpu_sc as plsc

NUM_EMBEDDINGS = 1000000
EMBEDDING_DIM = 64
BATCH = 16384
N_FIELDS = 26

NC = 2
NS = 16
NW = NC * NS

B = BATCH * N_FIELDS
GB = 128
CH = 4
G = GB * CH
B_PER_W = B // NW
N_GROUPS = B_PER_W // G
assert B_PER_W % G == 0


def _emb_body(idx_hbm, table_hbm, out_hbm, idx_v, rows_v, sem):
    wid = lax.axis_index("s") * NC + lax.axis_index("c")
    blk_base = wid * (B_PER_W // GB)

    def step(g, carry):
        blk = blk_base + g * CH
        pltpu.sync_copy(idx_hbm.at[pl.ds(blk, CH)], idx_v)
        descs = [
            pltpu.async_copy(
                table_hbm.at[idx_v.at[j]],
                rows_v.at[pl.ds(j * GB, GB)],
                sem,
            )
            for j in range(CH)
        ]
        for d in descs:
            d.wait()
        pltpu.sync_copy(rows_v, out_hbm.at[pl.ds(blk * GB, G)])
        return carry

    lax.fori_loop(0, N_GROUPS, step, 0)


@functools.partial(jax.jit, static_argnames=())
def kernel(x, embedding_weight):
    idx2d = x.astype(jnp.int32).reshape(B // GB, GB)
    mesh = plsc.VectorSubcoreMesh(
        core_axis_name="c", subcore_axis_name="s",
        num_cores=NC, num_subcores=NS,
    )
    out = pl.kernel(
        _emb_body,
        out_type=jax.ShapeDtypeStruct((B, EMBEDDING_DIM), jnp.float32),
        mesh=mesh,
        scratch_types=[
            pltpu.VMEM((CH, GB), jnp.int32),
            pltpu.VMEM((G, EMBEDDING_DIM), jnp.float32),
            pltpu.SemaphoreType.DMA,
        ],
        compiler_params=pltpu.CompilerParams(use_tc_tiling_on_sc=False),
    )(idx2d, embedding_weight)
    return out.reshape(BATCH, N_FIELDS, EMBEDDING_DIM)


# speedup vs baseline: 1.0321x; 1.0321x over previous
import functools

import jax
import jax.numpy as jnp
from jax import lax
from jax.experimental import pallas as pl
from jax.experimental.pallas import tpu as pltpu
from jax.experimental.pallas import tpu_sc as plsc

NUM_EMBEDDINGS = 1000000
EMBEDDING_DIM = 64
BATCH = 16384
N_FIELDS = 26

NC = 2
NS = 16
NW = NC * NS

B = BATCH * N_FIELDS
GB = 128
CH = 4
G = GB * CH
B_PER_W = B // NW
N_GROUPS = B_PER_W // G
NBUF = 3
assert B_PER_W % G == 0


def _emb_body(idx_hbm, table_hbm, out_hbm, idx_v, rows_v,
              sg0, sg1, sg2, so0, so1, so2):
    sem_g = [sg0, sg1, sg2]
    sem_o = [so0, so1, so2]
    wid = lax.axis_index("s") * NC + lax.axis_index("c")
    blk_base = wid * (B_PER_W // GB)

    def fire(g, b):
        blk = blk_base + g * CH
        pltpu.sync_copy(idx_hbm.at[pl.ds(blk, CH)], idx_v.at[b])
        for j in range(CH):
            pltpu.async_copy(
                table_hbm.at[idx_v.at[b].at[j]],
                rows_v.at[b].at[pl.ds(j * GB, GB)],
                sem_g[b],
            )

    def wait_gathers(b):
        for j in range(CH):
            pltpu.make_async_copy(
                table_hbm.at[idx_v.at[b].at[j]],
                rows_v.at[b].at[pl.ds(j * GB, GB)],
                sem_g[b],
            ).wait()

    def writeback(g, b):
        blk = blk_base + g * CH
        pltpu.async_copy(rows_v.at[b], out_hbm.at[pl.ds(blk * GB, G)], sem_o[b])

    def wait_writeback(g, b):
        blk = blk_base + g * CH
        pltpu.make_async_copy(
            rows_v.at[b], out_hbm.at[pl.ds(blk * GB, G)], sem_o[b]
        ).wait()

    fire(0, 0)
    fire(1, 1)
    wait_gathers(0)
    writeback(0, 0)
    fire(2, 2)
    wait_gathers(1)
    writeback(1, 1)

    def triple(t, carry):
        for k in range(NBUF):
            g = 3 * t + 3 + k
            b = k
            bp = (b + NBUF - 1) % NBUF
            wait_writeback(g - NBUF, b)
            fire(g, b)
            wait_gathers(bp)
            writeback(g - 1, bp)
        return carry

    lax.fori_loop(0, (N_GROUPS - 2 - NBUF) // NBUF, triple, 0)

    for g in (N_GROUPS - 2, N_GROUPS - 1):
        b = g % NBUF
        bp = (b + NBUF - 1) % NBUF
        wait_writeback(g - NBUF, b)
        fire(g, b)
        wait_gathers(bp)
        writeback(g - 1, bp)

    bl = (N_GROUPS - 1) % NBUF
    wait_gathers(bl)
    writeback(N_GROUPS - 1, bl)
    for g in (N_GROUPS - 3, N_GROUPS - 2, N_GROUPS - 1):
        wait_writeback(g, g % NBUF)


@functools.partial(jax.jit, static_argnames=())
def kernel(x, embedding_weight):
    idx2d = x.astype(jnp.int32).reshape(B // GB, GB)
    mesh = plsc.VectorSubcoreMesh(
        core_axis_name="c", subcore_axis_name="s",
        num_cores=NC, num_subcores=NS,
    )
    out = pl.kernel(
        _emb_body,
        out_type=jax.ShapeDtypeStruct((B, EMBEDDING_DIM), jnp.float32),
        mesh=mesh,
        scratch_types=[
            pltpu.VMEM((NBUF, CH, GB), jnp.int32),
            pltpu.VMEM((NBUF, G, EMBEDDING_DIM), jnp.float32),
            pltpu.SemaphoreType.DMA,
            pltpu.SemaphoreType.DMA,
            pltpu.SemaphoreType.DMA,
            pltpu.SemaphoreType.DMA,
            pltpu.SemaphoreType.DMA,
            pltpu.SemaphoreType.DMA,
        ],
        compiler_params=pltpu.CompilerParams(use_tc_tiling_on_sc=False),
    )(idx2d, embedding_weight)
    return out.reshape(BATCH, N_FIELDS, EMBEDDING_DIM)
